# baseline (speedup 1.0000x reference)
import functools

import jax
import jax.numpy as jnp
from jax import lax
from jax.experimental import pallas as pl
from jax.experimental.pallas import tpu as pltpu
from jax.experimental.pallas import tpu_sc as plsc

_N = 10000
_E = 320000
_NC = 2
_ND = 1024
_L = 50

_G = 32
_TILES = 16
_GPT = 640
_EPAD = _TILES * _G * _GPT
_NPAD = 10240
_RPT = _NPAD // _TILES
_LPAD = 64
_DPT = _ND // 32
_CG = 16
_NBUF = 8

_mesh = plsc.VectorSubcoreMesh(core_axis_name="c", subcore_axis_name="s")
_SC_PARAMS = pltpu.CompilerParams(needs_layout_passes=False)


def _make_segsum():
  out_type = [jax.ShapeDtypeStruct((_NPAD, 128), jnp.float32),
              jax.ShapeDtypeStruct((_NPAD, 128), jnp.float32)]
  scratch = [
      pltpu.VMEM((_CG, _G), jnp.int32),
      pltpu.VMEM((_CG, _G), jnp.int32),
      pltpu.VMEM((_NBUF, _G, 128), jnp.float32),
      pltpu.VMEM_SHARED((_NPAD, 128), jnp.float32),
      [pltpu.SemaphoreType.DMA] * _NBUF,
  ]

  def body(x0_hbm, x1_hbm, src_hbm, dst_hbm, zw_hbm,
           out0, out1, src_v, dst_v, rows_v, agg_sh, sems):
    cid = lax.axis_index("c")
    sid = lax.axis_index("s")
    r0 = sid * _RPT
    g0 = sid * _GPT
    pltpu.sync_copy(zw_hbm.at[pl.ds(r0, _RPT)], agg_sh.at[pl.ds(r0, _RPT)])
    plsc.subcore_barrier()

    def run(x_ref):
      def chunk(c, carry):
        pltpu.sync_copy(src_hbm.at[pl.ds(g0 + c * _CG, _CG)], src_v)
        pltpu.sync_copy(dst_hbm.at[pl.ds(g0 + c * _CG, _CG)], dst_v)
        for g in range(_NBUF):
          pltpu.async_copy(x_ref.at[src_v.at[g]], rows_v.at[g], sems[g])
        for g in range(_CG):
          b = g % _NBUF
          pltpu.make_async_copy(x_ref.at[src_v.at[g]], rows_v.at[b],
                                sems[b]).wait()
          pltpu.sync_copy(rows_v.at[b], agg_sh.at[dst_v.at[g]], add=True)
          if g + _NBUF < _CG:
            pltpu.async_copy(x_ref.at[src_v.at[g + _NBUF]], rows_v.at[b],
                             sems[b])
        return carry
      lax.fori_loop(0, _GPT // _CG, chunk, 0)

    @pl.when(cid == 0)
    def _():
      run(x0_hbm)

    @pl.when(cid == 1)
    def _():
      run(x1_hbm)

    plsc.subcore_barrier()

    @pl.when(cid == 0)
    def _():
      pltpu.sync_copy(agg_sh.at[pl.ds(r0, _RPT)], out0.at[pl.ds(r0, _RPT)])

    @pl.when(cid == 1)
    def _():
      pltpu.sync_copy(agg_sh.at[pl.ds(r0, _RPT)], out1.at[pl.ds(r0, _RPT)])

  return pl.kernel(body, out_type=out_type, mesh=_mesh,
                   scratch_types=scratch, compiler_params=_SC_PARAMS)


def _make_segsum_l1():
  half_g = _GPT // 2
  drows = _NPAD // 128
  out_type = [jax.ShapeDtypeStruct((_NPAD, 128), jnp.float32),
              jax.ShapeDtypeStruct((_NPAD, 128), jnp.float32),
              jax.ShapeDtypeStruct((drows, 128), jnp.float32),
              jax.ShapeDtypeStruct((drows, 128), jnp.float32)]
  scratch = [
      pltpu.VMEM((_CG, _G), jnp.int32),
      pltpu.VMEM((_CG, _G), jnp.int32),
      pltpu.VMEM((_NBUF, _G, 128), jnp.float32),
      pltpu.VMEM((drows, 128), jnp.float32),
      pltpu.VMEM((drows,), jnp.int32),
      pltpu.VMEM_SHARED((_NPAD, 128), jnp.float32),
      pltpu.VMEM_SHARED((drows, 128), jnp.float32),
      [pltpu.SemaphoreType.DMA] * _NBUF,
  ]

  def body(x_hbm, src_hbm, dst_hbm, zw_hbm, iota_hbm,
           out0, out1, deg0, deg1,
           src_v, dst_v, rows_v, degacc_v, iota_v,
           agg_sh, deg_sh, sems):
    cid = lax.axis_index("c")
    sid = lax.axis_index("s")
    r0 = sid * _RPT
    g0 = cid * (half_g * _TILES) + sid * half_g
    pltpu.sync_copy(zw_hbm.at[pl.ds(r0, _RPT)], agg_sh.at[pl.ds(r0, _RPT)])
    pltpu.sync_copy(zw_hbm.at[pl.ds(0, drows)], degacc_v)
    pltpu.sync_copy(iota_hbm, iota_v)

    @pl.when(sid == 0)
    def _():
      pltpu.sync_copy(zw_hbm.at[pl.ds(0, drows)], deg_sh)

    plsc.subcore_barrier()

    ones = jnp.ones((16,), jnp.float32)

    vpr = _G // 16

    def chunk(c, carry):
      pltpu.sync_copy(src_hbm.at[pl.ds(g0 + c * _CG, _CG)], src_v)
      pltpu.sync_copy(dst_hbm.at[pl.ds(g0 + c * _CG, _CG)], dst_v)
      for g in range(_NBUF):
        pltpu.async_copy(x_hbm.at[src_v.at[g]], rows_v.at[g], sems[g])
      for g in range(_CG):
        b = g % _NBUF
        pltpu.make_async_copy(x_hbm.at[src_v.at[g]], rows_v.at[b],
                              sems[b]).wait()
        pltpu.sync_copy(rows_v.at[b], agg_sh.at[dst_v.at[g]], add=True)
        if g + _NBUF < _CG:
          pltpu.async_copy(x_hbm.at[src_v.at[g + _NBUF]], rows_v.at[b],
                           sems[b])

      def dstep2(i, carry2):
        d = dst_v[i // vpr, pl.ds((i % vpr) * 16, 16)]
        plsc.addupdate_scatter(degacc_v, [d // 128, d % 128], ones)
        return carry2

      lax.fori_loop(0, _CG * _G // 16, dstep2, 0)
      return carry

    lax.fori_loop(0, half_g // _CG, chunk, 0)
    pltpu.sync_copy(degacc_v, deg_sh.at[iota_v], add=True)
    plsc.subcore_barrier()

    @pl.when(cid == 0)
    def _():
      pltpu.sync_copy(agg_sh.at[pl.ds(r0, _RPT)], out0.at[pl.ds(r0, _RPT)])

      @pl.when(sid == 0)
      def _():
        pltpu.sync_copy(deg_sh, deg0)

    @pl.when(cid == 1)
    def _():
      pltpu.sync_copy(agg_sh.at[pl.ds(r0, _RPT)], out1.at[pl.ds(r0, _RPT)])

      @pl.when(sid == 0)
      def _():
        pltpu.sync_copy(deg_sh, deg1)

  return pl.kernel(body, out_type=out_type, mesh=_mesh,
                   scratch_types=scratch, compiler_params=_SC_PARAMS)


def _graph_norm_relu(h, g, be, a):
  mean = jnp.mean(h, axis=0, keepdims=True)
  sub = h - a * mean
  var = jnp.mean(sub * sub, axis=0, keepdims=True)
  return g * sub / jnp.sqrt(var + 1e-5) + be


def _tc_layer1(p0, p1, d0, d1, x, wlt, bl, wrt, g, be, a):
  out_shape = [jax.ShapeDtypeStruct((_N, 128), jnp.float32),
               jax.ShapeDtypeStruct((_N, 128), jnp.float32),
               jax.ShapeDtypeStruct((_N, 1), jnp.float32)]

  def body(p0_r, p1_r, d0_r, d1_r, x_r, wl_r, bl_r, wr_r, g_r, be_r, a_r,
           o0, o1, sc_r):
    scale = 1.0 / jnp.clip(d0_r[...] + d1_r[...], 1.0, None)
    agg = (p0_r[...][:_N] + p1_r[...][:_N]) * scale
    h = (jnp.dot(agg, wl_r[...], preferred_element_type=jnp.float32)
         + jnp.dot(x_r[...], wr_r[...], preferred_element_type=jnp.float32)
         + bl_r[...])
    gn = _graph_norm_relu(h, g_r[...], be_r[...], a_r[...])
    o0[...] = jnp.maximum(gn[:, :128], 0.0)
    o1[...] = jnp.maximum(gn[:, 128:], 0.0)
    sc_r[...] = scale

  return pl.pallas_call(body, out_shape=out_shape)(
      p0, p1, d0, d1, x, wlt, bl, wrt, g, be, a)


def _tc_layer(agg0, agg1, scale, x0, x1, wl0, wl1, bl, wr0, wr1, g, be, a,
              fcwt0=None, fcwt1=None):
  last = fcwt0 is not None
  if last:
    out_shape = [jax.ShapeDtypeStruct((_N, _NC), jnp.float32)]
  else:
    out_shape = [jax.ShapeDtypeStruct((_N, 128), jnp.float32),
                 jax.ShapeDtypeStruct((_N, 128), jnp.float32)]

  def body(agg0_r, agg1_r, sc_r, x0_r, x1_r, wl0_r, wl1_r, bl_r,
           wr0_r, wr1_r, g_r, be_r, a_r, *rest):
    scale = sc_r[...]
    a0 = agg0_r[...][:_N] * scale
    a1 = agg1_r[...][:_N] * scale
    xx0 = x0_r[...]
    xx1 = x1_r[...]
    h = (jnp.dot(a0, wl0_r[...], preferred_element_type=jnp.float32)
         + jnp.dot(a1, wl1_r[...], preferred_element_type=jnp.float32)
         + jnp.dot(xx0, wr0_r[...], preferred_element_type=jnp.float32)
         + jnp.dot(xx1, wr1_r[...], preferred_element_type=jnp.float32)
         + bl_r[...])
    gn = _graph_norm_relu(h, g_r[...], be_r[...], a_r[...])
    act0 = jnp.maximum(gn[:, :128] + xx0, 0.0)
    act1 = jnp.maximum(gn[:, 128:] + xx1, 0.0)
    if last:
      fw0_r, fw1_r, z_r = rest
      z_r[...] = (
          jnp.dot(act0, fw0_r[...], preferred_element_type=jnp.float32)
          + jnp.dot(act1, fw1_r[...], preferred_element_type=jnp.float32))
    else:
      rest[0][...] = act0
      rest[1][...] = act1

  args = [agg0, agg1, scale, x0, x1, wl0, wl1, bl, wr0, wr1, g, be, a]
  if last:
    args += [fcwt0, fcwt1]
  return pl.pallas_call(
      body, out_shape=out_shape,
      compiler_params=pltpu.CompilerParams(
          vmem_limit_bytes=100 * 1024 * 1024))(*args)


def _make_pool():
  out_type = jax.ShapeDtypeStruct((_NC * _ND,), jnp.float32)
  scratch = [
      pltpu.VMEM((_N * _NC,), jnp.float32),
      pltpu.VMEM((_DPT * _LPAD,), jnp.int32),
      pltpu.VMEM((_DPT * _LPAD,), jnp.float32),
      pltpu.VMEM((2 * 16,), jnp.float32),
      pltpu.VMEM((_NC * _DPT,), jnp.float32),
  ]

  def body(z_hbm, ids_hbm, w_hbm, fcb_hbm, out_hbm,
           z_v, ids_v, w_v, fcb_v, out_v):
    cid = lax.axis_index("c")
    sid = lax.axis_index("s")
    wid = sid * 2 + cid
    pltpu.sync_copy(z_hbm, z_v)
    pltpu.sync_copy(ids_hbm.at[pl.ds(wid * _DPT * _LPAD, _DPT * _LPAD)],
                    ids_v)
    pltpu.sync_copy(w_hbm.at[pl.ds(wid * _DPT * _LPAD, _DPT * _LPAD)], w_v)
    pltpu.sync_copy(fcb_hbm, fcb_v)
    fcb0 = fcb_v[pl.ds(0, 16)]
    fcb1 = fcb_v[pl.ds(16, 16)]
    for g in range(_DPT // 16):
      acc0 = jnp.zeros((16,), jnp.float32)
      acc1 = jnp.zeros((16,), jnp.float32)
      wacc = jnp.zeros((16,), jnp.float32)
      for l in range(_LPAD):
        off = l * _DPT + g * 16
        idx = ids_v[pl.ds(off, 16)]
        w = w_v[pl.ds(off, 16)]
        z0 = plsc.load_gather(z_v, [idx * 2])
        z1 = plsc.load_gather(z_v, [idx * 2 + 1])
        acc0 = acc0 + w * z0
        acc1 = acc1 + w * z1
        wacc = wacc + w
      inv = 1.0 / (wacc + 1e-8)
      out_v[pl.ds(g * 16, 16)] = acc0 * inv + fcb0
      out_v[pl.ds(_DPT + g * 16, 16)] = acc1 * inv + fcb1
    pltpu.sync_copy(out_v.at[pl.ds(0, _DPT)],
                    out_hbm.at[pl.ds(wid * _DPT, _DPT)])
    pltpu.sync_copy(out_v.at[pl.ds(_DPT, _DPT)],
                    out_hbm.at[pl.ds(_ND + wid * _DPT, _DPT)])

  return pl.kernel(body, out_type=out_type, mesh=_mesh,
                   scratch_types=scratch, compiler_params=_SC_PARAMS)


_segsum_l1 = _make_segsum_l1()
_segsum = _make_segsum()
_pool = _make_pool()


def kernel(x, edge_index, doc_word_ids, doc_weights,
           W1l, b1l, W1r, g1, be1, a1,
           W2l, b2l, W2r, g2, be2, a2,
           W3l, b3l, W3r, g3, be3, a3,
           W4l, b4l, W4r, g4, be4, a4,
           fcW, fcb):
  src = edge_index[0].astype(jnp.int32)
  dst = edge_index[1].astype(jnp.int32)
  pad_e = _EPAD - _E
  src_p = jnp.concatenate(
      [src, jnp.zeros((pad_e,), jnp.int32)]).reshape(_EPAD // _G, _G)
  dst_f = jnp.concatenate([dst, jnp.full((pad_e,), _N, jnp.int32)])
  dst_p = dst_f.reshape(_EPAD // _G, _G)
  z128 = jnp.zeros((_NPAD, 128), jnp.float32)
  iota = jnp.arange(_NPAD // 128, dtype=jnp.int32)

  def halves(w):
    wt = w.T
    h = wt.shape[0] // 2
    return wt[:h], wt[h:]

  p0, p1, deg0, deg1 = _segsum_l1(x, src_p, dst_p, z128, iota)
  d0 = deg0.reshape(-1, 1)[:_N]
  d1 = deg1.reshape(-1, 1)[:_N]
  h0, h1, scale = _tc_layer1(p0, p1, d0, d1, x, W1l.T, b1l[None],
                             W1r.T, g1[None], be1[None], a1[None])
  for Wl, bl, Wr, g, be, a in ((W2l, b2l, W2r, g2, be2, a2),
                               (W3l, b3l, W3r, g3, be3, a3)):
    agg0, agg1 = _segsum(h0, h1, src_p, dst_p, z128)
    wl0, wl1 = halves(Wl)
    wr0, wr1 = halves(Wr)
    h0, h1 = _tc_layer(agg0, agg1, scale, h0, h1, wl0, wl1, bl[None],
                       wr0, wr1, g[None], be[None], a[None])
  agg0, agg1 = _segsum(h0, h1, src_p, dst_p, z128)
  wl0, wl1 = halves(W4l)
  wr0, wr1 = halves(W4r)
  fw0, fw1 = halves(fcW)
  z = _tc_layer(agg0, agg1, scale, h0, h1, wl0, wl1, b4l[None],
                wr0, wr1, g4[None], be4[None], a4[None],
                fcwt0=fw0, fcwt1=fw1)[0]

  def tileize(arr):
    p = jnp.pad(arr, ((0, 0), (0, _LPAD - _L)))
    return p.T.reshape(_LPAD, 32, _DPT).transpose(1, 0, 2).reshape(-1)

  ids_f = tileize(doc_word_ids.astype(jnp.int32))
  w_f = tileize(doc_weights)
  out_f = _pool(z.reshape(-1), ids_f, w_f, jnp.repeat(fcb, 16))
  return out_f.reshape(_NC, _ND).T


# speedup vs baseline: 1.1292x; 1.1292x over previous
import functools

import jax
import jax.numpy as jnp
from jax import lax
from jax.experimental import pallas as pl
from jax.experimental.pallas import tpu as pltpu
from jax.experimental.pallas import tpu_sc as plsc

_N = 10000
_E = 320000
_NC = 2
_ND = 1024
_L = 50

_G = 80
_TILES = 16
_GPT = 256
_EPAD = _TILES * _G * _GPT
_G1 = 64
_GPT1 = _EPAD // (_TILES * _G1)
_NPAD = 10240
_RPT = _NPAD // _TILES
_LPAD = 64
_DPT = _ND // 32
_CG = 16
_NBUF = 4

_mesh = plsc.VectorSubcoreMesh(core_axis_name="c", subcore_axis_name="s")
_SC_PARAMS = pltpu.CompilerParams(needs_layout_passes=False)


def _make_segsum():
  out_type = [jax.ShapeDtypeStruct((_NPAD, 128), jnp.float32),
              jax.ShapeDtypeStruct((_NPAD, 128), jnp.float32)]
  scratch = [
      pltpu.VMEM((_CG, _G), jnp.int32),
      pltpu.VMEM((_CG, _G), jnp.int32),
      pltpu.VMEM((_NBUF, _G, 128), jnp.float32),
      pltpu.VMEM_SHARED((_NPAD, 128), jnp.float32),
      [pltpu.SemaphoreType.DMA] * _NBUF,
  ]

  def body(x0_hbm, x1_hbm, src_hbm, dst_hbm, zw_hbm,
           out0, out1, src_v, dst_v, rows_v, agg_sh, sems):
    cid = lax.axis_index("c")
    sid = lax.axis_index("s")
    r0 = sid * _RPT
    g0 = sid * _GPT
    pltpu.sync_copy(zw_hbm.at[pl.ds(r0, _RPT)], agg_sh.at[pl.ds(r0, _RPT)])
    plsc.subcore_barrier()

    def run(x_ref):
      def chunk(c, carry):
        pltpu.sync_copy(src_hbm.at[pl.ds(g0 + c * _CG, _CG)], src_v)
        pltpu.sync_copy(dst_hbm.at[pl.ds(g0 + c * _CG, _CG)], dst_v)
        for g in range(_NBUF):
          pltpu.async_copy(x_ref.at[src_v.at[g]], rows_v.at[g], sems[g])
        for g in range(_CG):
          b = g % _NBUF
          pltpu.make_async_copy(x_ref.at[src_v.at[g]], rows_v.at[b],
                                sems[b]).wait()
          pltpu.sync_copy(rows_v.at[b], agg_sh.at[dst_v.at[g]], add=True)
          if g + _NBUF < _CG:
            pltpu.async_copy(x_ref.at[src_v.at[g + _NBUF]], rows_v.at[b],
                             sems[b])
        return carry
      lax.fori_loop(0, _GPT // _CG, chunk, 0)

    @pl.when(cid == 0)
    def _():
      run(x0_hbm)

    @pl.when(cid == 1)
    def _():
      run(x1_hbm)

    plsc.subcore_barrier()

    @pl.when(cid == 0)
    def _():
      pltpu.sync_copy(agg_sh.at[pl.ds(r0, _RPT)], out0.at[pl.ds(r0, _RPT)])

    @pl.when(cid == 1)
    def _():
      pltpu.sync_copy(agg_sh.at[pl.ds(r0, _RPT)], out1.at[pl.ds(r0, _RPT)])

  return pl.kernel(body, out_type=out_type, mesh=_mesh,
                   scratch_types=scratch, compiler_params=_SC_PARAMS)


def _make_segsum_l1():
  half_g = _GPT1 // 2
  drows = _NPAD // 128
  out_type = [jax.ShapeDtypeStruct((_NPAD, 128), jnp.float32),
              jax.ShapeDtypeStruct((_NPAD, 128), jnp.float32),
              jax.ShapeDtypeStruct((drows, 128), jnp.float32),
              jax.ShapeDtypeStruct((drows, 128), jnp.float32)]
  scratch = [
      pltpu.VMEM((_CG, _G1), jnp.int32),
      pltpu.VMEM((_CG, _G1), jnp.int32),
      pltpu.VMEM((_NBUF, _G1, 128), jnp.float32),
      pltpu.VMEM((drows, 128), jnp.float32),
      pltpu.VMEM((drows,), jnp.int32),
      pltpu.VMEM_SHARED((_NPAD, 128), jnp.float32),
      pltpu.VMEM_SHARED((drows, 128), jnp.float32),
      [pltpu.SemaphoreType.DMA] * _NBUF,
  ]

  def body(x_hbm, src_hbm, dst_hbm, zw_hbm, iota_hbm,
           out0, out1, deg0, deg1,
           src_v, dst_v, rows_v, degacc_v, iota_v,
           agg_sh, deg_sh, sems):
    cid = lax.axis_index("c")
    sid = lax.axis_index("s")
    r0 = sid * _RPT
    g0 = cid * (half_g * _TILES) + sid * half_g
    pltpu.sync_copy(zw_hbm.at[pl.ds(r0, _RPT)], agg_sh.at[pl.ds(r0, _RPT)])
    pltpu.sync_copy(zw_hbm.at[pl.ds(0, drows)], degacc_v)
    pltpu.sync_copy(iota_hbm, iota_v)

    @pl.when(sid == 0)
    def _():
      pltpu.sync_copy(zw_hbm.at[pl.ds(0, drows)], deg_sh)

    plsc.subcore_barrier()

    ones = jnp.ones((16,), jnp.float32)

    vpr = _G1 // 16

    def chunk(c, carry):
      pltpu.sync_copy(src_hbm.at[pl.ds(g0 + c * _CG, _CG)], src_v)
      pltpu.sync_copy(dst_hbm.at[pl.ds(g0 + c * _CG, _CG)], dst_v)
      for g in range(_NBUF):
        pltpu.async_copy(x_hbm.at[src_v.at[g]], rows_v.at[g], sems[g])
      for g in range(_CG):
        b = g % _NBUF
        pltpu.make_async_copy(x_hbm.at[src_v.at[g]], rows_v.at[b],
                              sems[b]).wait()
        pltpu.sync_copy(rows_v.at[b], agg_sh.at[dst_v.at[g]], add=True)
        if g + _NBUF < _CG:
          pltpu.async_copy(x_hbm.at[src_v.at[g + _NBUF]], rows_v.at[b],
                           sems[b])

      def dstep2(i, carry2):
        d = dst_v[i // vpr, pl.ds((i % vpr) * 16, 16)]
        plsc.addupdate_scatter(degacc_v, [d // 128, d % 128], ones)
        return carry2

      lax.fori_loop(0, _CG * _G1 // 16, dstep2, 0)
      return carry

    lax.fori_loop(0, half_g // _CG, chunk, 0)
    pltpu.sync_copy(degacc_v, deg_sh.at[iota_v], add=True)
    plsc.subcore_barrier()

    @pl.when(cid == 0)
    def _():
      pltpu.sync_copy(agg_sh.at[pl.ds(r0, _RPT)], out0.at[pl.ds(r0, _RPT)])

      @pl.when(sid == 0)
      def _():
        pltpu.sync_copy(deg_sh, deg0)

    @pl.when(cid == 1)
    def _():
      pltpu.sync_copy(agg_sh.at[pl.ds(r0, _RPT)], out1.at[pl.ds(r0, _RPT)])

      @pl.when(sid == 0)
      def _():
        pltpu.sync_copy(deg_sh, deg1)

  return pl.kernel(body, out_type=out_type, mesh=_mesh,
                   scratch_types=scratch, compiler_params=_SC_PARAMS)


def _graph_norm_relu(h, g, be, a):
  mean = jnp.mean(h, axis=0, keepdims=True)
  sub = h - a * mean
  var = jnp.mean(sub * sub, axis=0, keepdims=True)
  return g * sub / jnp.sqrt(var + 1e-5) + be


def _tc_layer1(p0, p1, d0, d1, x, wlt, bl, wrt, g, be, a):
  out_shape = [jax.ShapeDtypeStruct((_N, 128), jnp.float32),
               jax.ShapeDtypeStruct((_N, 128), jnp.float32),
               jax.ShapeDtypeStruct((_N, 1), jnp.float32)]

  def body(p0_r, p1_r, d0_r, d1_r, x_r, wl_r, bl_r, wr_r, g_r, be_r, a_r,
           o0, o1, sc_r):
    scale = 1.0 / jnp.clip(d0_r[...] + d1_r[...], 1.0, None)
    agg = (p0_r[...][:_N] + p1_r[...][:_N]) * scale
    h = (jnp.dot(agg, wl_r[...], preferred_element_type=jnp.float32)
         + jnp.dot(x_r[...], wr_r[...], preferred_element_type=jnp.float32)
         + bl_r[...])
    gn = _graph_norm_relu(h, g_r[...], be_r[...], a_r[...])
    o0[...] = jnp.maximum(gn[:, :128], 0.0)
    o1[...] = jnp.maximum(gn[:, 128:], 0.0)
    sc_r[...] = scale

  return pl.pallas_call(body, out_shape=out_shape)(
      p0, p1, d0, d1, x, wlt, bl, wrt, g, be, a)


def _tc_layer(agg0, agg1, scale, x0, x1, wl0, wl1, bl, wr0, wr1, g, be, a,
              fcwt0=None, fcwt1=None):
  last = fcwt0 is not None
  if last:
    out_shape = [jax.ShapeDtypeStruct((_N, _NC), jnp.float32)]
  else:
    out_shape = [jax.ShapeDtypeStruct((_N, 128), jnp.float32),
                 jax.ShapeDtypeStruct((_N, 128), jnp.float32)]

  def body(agg0_r, agg1_r, sc_r, x0_r, x1_r, wl0_r, wl1_r, bl_r,
           wr0_r, wr1_r, g_r, be_r, a_r, *rest):
    scale = sc_r[...]
    a0 = agg0_r[...][:_N] * scale
    a1 = agg1_r[...][:_N] * scale
    xx0 = x0_r[...]
    xx1 = x1_r[...]
    h = (jnp.dot(a0, wl0_r[...], preferred_element_type=jnp.float32)
         + jnp.dot(a1, wl1_r[...], preferred_element_type=jnp.float32)
         + jnp.dot(xx0, wr0_r[...], preferred_element_type=jnp.float32)
         + jnp.dot(xx1, wr1_r[...], preferred_element_type=jnp.float32)
         + bl_r[...])
    gn = _graph_norm_relu(h, g_r[...], be_r[...], a_r[...])
    act0 = jnp.maximum(gn[:, :128] + xx0, 0.0)
    act1 = jnp.maximum(gn[:, 128:] + xx1, 0.0)
    if last:
      fw0_r, fw1_r, z_r = rest
      z_r[...] = (
          jnp.dot(act0, fw0_r[...], preferred_element_type=jnp.float32)
          + jnp.dot(act1, fw1_r[...], preferred_element_type=jnp.float32))
    else:
      rest[0][...] = act0
      rest[1][...] = act1

  args = [agg0, agg1, scale, x0, x1, wl0, wl1, bl, wr0, wr1, g, be, a]
  if last:
    args += [fcwt0, fcwt1]
  return pl.pallas_call(
      body, out_shape=out_shape,
      compiler_params=pltpu.CompilerParams(
          vmem_limit_bytes=100 * 1024 * 1024))(*args)


def _make_pool():
  out_type = jax.ShapeDtypeStruct((_NC * _ND,), jnp.float32)
  scratch = [
      pltpu.VMEM((_N * _NC,), jnp.float32),
      pltpu.VMEM((_DPT * _LPAD,), jnp.int32),
      pltpu.VMEM((_DPT * _LPAD,), jnp.float32),
      pltpu.VMEM((2 * 16,), jnp.float32),
      pltpu.VMEM((_NC * _DPT,), jnp.float32),
  ]

  def body(z_hbm, ids_hbm, w_hbm, fcb_hbm, out_hbm,
           z_v, ids_v, w_v, fcb_v, out_v):
    cid = lax.axis_index("c")
    sid = lax.axis_index("s")
    wid = sid * 2 + cid
    pltpu.sync_copy(z_hbm, z_v)
    pltpu.sync_copy(ids_hbm.at[pl.ds(wid * _DPT * _LPAD, _DPT * _LPAD)],
                    ids_v)
    pltpu.sync_copy(w_hbm.at[pl.ds(wid * _DPT * _LPAD, _DPT * _LPAD)], w_v)
    pltpu.sync_copy(fcb_hbm, fcb_v)
    fcb0 = fcb_v[pl.ds(0, 16)]
    fcb1 = fcb_v[pl.ds(16, 16)]
    for g in range(_DPT // 16):
      acc0 = jnp.zeros((16,), jnp.float32)
      acc1 = jnp.zeros((16,), jnp.float32)
      wacc = jnp.zeros((16,), jnp.float32)
      for l in range(_LPAD):
        off = l * _DPT + g * 16
        idx = ids_v[pl.ds(off, 16)]
        w = w_v[pl.ds(off, 16)]
        z0 = plsc.load_gather(z_v, [idx * 2])
        z1 = plsc.load_gather(z_v, [idx * 2 + 1])
        acc0 = acc0 + w * z0
        acc1 = acc1 + w * z1
        wacc = wacc + w
      inv = 1.0 / (wacc + 1e-8)
      out_v[pl.ds(g * 16, 16)] = acc0 * inv + fcb0
      out_v[pl.ds(_DPT + g * 16, 16)] = acc1 * inv + fcb1
    pltpu.sync_copy(out_v.at[pl.ds(0, _DPT)],
                    out_hbm.at[pl.ds(wid * _DPT, _DPT)])
    pltpu.sync_copy(out_v.at[pl.ds(_DPT, _DPT)],
                    out_hbm.at[pl.ds(_ND + wid * _DPT, _DPT)])

  return pl.kernel(body, out_type=out_type, mesh=_mesh,
                   scratch_types=scratch, compiler_params=_SC_PARAMS)


_segsum_l1 = _make_segsum_l1()
_segsum = _make_segsum()
_pool = _make_pool()


def kernel(x, edge_index, doc_word_ids, doc_weights,
           W1l, b1l, W1r, g1, be1, a1,
           W2l, b2l, W2r, g2, be2, a2,
           W3l, b3l, W3r, g3, be3, a3,
           W4l, b4l, W4r, g4, be4, a4,
           fcW, fcb):
  src = edge_index[0].astype(jnp.int32)
  dst = edge_index[1].astype(jnp.int32)
  pad_e = _EPAD - _E
  src_f = jnp.concatenate([src, jnp.zeros((pad_e,), jnp.int32)])
  dst_f = jnp.concatenate([dst, jnp.full((pad_e,), _N, jnp.int32)])
  src_p = src_f.reshape(_EPAD // _G, _G)
  dst_p = dst_f.reshape(_EPAD // _G, _G)
  src_p1 = src_f.reshape(_EPAD // _G1, _G1)
  dst_p1 = dst_f.reshape(_EPAD // _G1, _G1)
  z128 = jnp.zeros((_NPAD, 128), jnp.float32)
  iota = jnp.arange(_NPAD // 128, dtype=jnp.int32)

  def halves(w):
    wt = w.T
    h = wt.shape[0] // 2
    return wt[:h], wt[h:]

  p0, p1, deg0, deg1 = _segsum_l1(x, src_p1, dst_p1, z128, iota)
  d0 = deg0.reshape(-1, 1)[:_N]
  d1 = deg1.reshape(-1, 1)[:_N]
  h0, h1, scale = _tc_layer1(p0, p1, d0, d1, x, W1l.T, b1l[None],
                             W1r.T, g1[None], be1[None], a1[None])
  for Wl, bl, Wr, g, be, a in ((W2l, b2l, W2r, g2, be2, a2),
                               (W3l, b3l, W3r, g3, be3, a3)):
    agg0, agg1 = _segsum(h0, h1, src_p, dst_p, z128)
    wl0, wl1 = halves(Wl)
    wr0, wr1 = halves(Wr)
    h0, h1 = _tc_layer(agg0, agg1, scale, h0, h1, wl0, wl1, bl[None],
                       wr0, wr1, g[None], be[None], a[None])
  agg0, agg1 = _segsum(h0, h1, src_p, dst_p, z128)
  wl0, wl1 = halves(W4l)
  wr0, wr1 = halves(W4r)
  fw0, fw1 = halves(fcW)
  z = _tc_layer(agg0, agg1, scale, h0, h1, wl0, wl1, b4l[None],
                wr0, wr1, g4[None], be4[None], a4[None],
                fcwt0=fw0, fcwt1=fw1)[0]

  def tileize(arr):
    p = jnp.pad(arr, ((0, 0), (0, _LPAD - _L)))
    return p.T.reshape(_LPAD, 32, _DPT).transpose(1, 0, 2).reshape(-1)

  ids_f = tileize(doc_word_ids.astype(jnp.int32))
  w_f = tileize(doc_weights)
  out_f = _pool(z.reshape(-1), ids_f, w_f, jnp.repeat(fcb, 16))
  return out_f.reshape(_NC, _ND).T


# speedup vs baseline: 1.1353x; 1.0054x over previous
import functools

import jax
import jax.numpy as jnp
from jax import lax
from jax.experimental import pallas as pl
from jax.experimental.pallas import tpu as pltpu
from jax.experimental.pallas import tpu_sc as plsc

_N = 10000
_E = 320000
_NC = 2
_ND = 1024
_L = 50

_G = 80
_TILES = 16
_GPT = 256
_EPAD = _TILES * _G * _GPT
_G1 = 64
_GPT1 = _EPAD // (_TILES * _G1)
_NPAD = 10240
_RPT = _NPAD // _TILES
_LPAD = 64
_DPT = _ND // 32
_CG = 16
_NBUF = 4

_mesh = plsc.VectorSubcoreMesh(core_axis_name="c", subcore_axis_name="s")
_SC_PARAMS = pltpu.CompilerParams(needs_layout_passes=False)


def _make_segsum():
  out_type = [jax.ShapeDtypeStruct((_NPAD, 128), jnp.float32),
              jax.ShapeDtypeStruct((_NPAD, 128), jnp.float32)]
  scratch = [
      pltpu.VMEM((_CG, _G), jnp.int32),
      pltpu.VMEM((_CG, _G), jnp.int32),
      pltpu.VMEM((_NBUF, _G, 128), jnp.float32),
      pltpu.VMEM_SHARED((_NPAD, 128), jnp.float32),
      [pltpu.SemaphoreType.DMA] * _NBUF,
  ]

  def body(x0_hbm, x1_hbm, src_hbm, dst_hbm, zw_hbm,
           out0, out1, src_v, dst_v, rows_v, agg_sh, sems):
    cid = lax.axis_index("c")
    sid = lax.axis_index("s")
    r0 = sid * _RPT
    g0 = sid * _GPT
    pltpu.sync_copy(zw_hbm.at[pl.ds(r0, _RPT)], agg_sh.at[pl.ds(r0, _RPT)])
    plsc.subcore_barrier()

    def run(x_ref):
      def chunk(c, carry):
        pltpu.sync_copy(src_hbm.at[pl.ds(g0 + c * _CG, _CG)], src_v)
        pltpu.sync_copy(dst_hbm.at[pl.ds(g0 + c * _CG, _CG)], dst_v)
        for g in range(_NBUF):
          pltpu.async_copy(x_ref.at[src_v.at[g]], rows_v.at[g], sems[g])
        for g in range(_CG):
          b = g % _NBUF
          pltpu.make_async_copy(x_ref.at[src_v.at[g]], rows_v.at[b],
                                sems[b]).wait()
          pltpu.sync_copy(rows_v.at[b], agg_sh.at[dst_v.at[g]], add=True)
          if g + _NBUF < _CG:
            pltpu.async_copy(x_ref.at[src_v.at[g + _NBUF]], rows_v.at[b],
                             sems[b])
        return carry
      lax.fori_loop(0, _GPT // _CG, chunk, 0)

    @pl.when(cid == 0)
    def _():
      run(x0_hbm)

    @pl.when(cid == 1)
    def _():
      run(x1_hbm)

    plsc.subcore_barrier()

    @pl.when(cid == 0)
    def _():
      pltpu.sync_copy(agg_sh.at[pl.ds(r0, _RPT)], out0.at[pl.ds(r0, _RPT)])

    @pl.when(cid == 1)
    def _():
      pltpu.sync_copy(agg_sh.at[pl.ds(r0, _RPT)], out1.at[pl.ds(r0, _RPT)])

  return pl.kernel(body, out_type=out_type, mesh=_mesh,
                   scratch_types=scratch, compiler_params=_SC_PARAMS)


def _make_segsum_l1():
  half_g = _GPT1 // 2
  drows = _NPAD // 128
  out_type = [jax.ShapeDtypeStruct((_NPAD, 128), jnp.float32),
              jax.ShapeDtypeStruct((_NPAD, 128), jnp.float32),
              jax.ShapeDtypeStruct((drows, 128), jnp.float32),
              jax.ShapeDtypeStruct((drows, 128), jnp.float32)]
  scratch = [
      pltpu.VMEM((_CG, _G1), jnp.int32),
      pltpu.VMEM((_CG, _G1), jnp.int32),
      pltpu.VMEM((_NBUF, _G1, 128), jnp.float32),
      pltpu.VMEM((drows, 128), jnp.float32),
      pltpu.VMEM((drows,), jnp.int32),
      pltpu.VMEM_SHARED((_NPAD, 128), jnp.float32),
      pltpu.VMEM_SHARED((drows, 128), jnp.float32),
      [pltpu.SemaphoreType.DMA] * _NBUF,
  ]

  def body(x_hbm, src_hbm, dst_hbm, zw_hbm, iota_hbm,
           out0, out1, deg0, deg1,
           src_v, dst_v, rows_v, degacc_v, iota_v,
           agg_sh, deg_sh, sems):
    cid = lax.axis_index("c")
    sid = lax.axis_index("s")
    r0 = sid * _RPT
    g0 = cid * (half_g * _TILES) + sid * half_g
    pltpu.sync_copy(zw_hbm.at[pl.ds(r0, _RPT)], agg_sh.at[pl.ds(r0, _RPT)])
    pltpu.sync_copy(zw_hbm.at[pl.ds(0, drows)], degacc_v)
    pltpu.sync_copy(iota_hbm, iota_v)

    @pl.when(sid == 0)
    def _():
      pltpu.sync_copy(zw_hbm.at[pl.ds(0, drows)], deg_sh)

    plsc.subcore_barrier()

    ones = jnp.ones((16,), jnp.float32)

    vpr = _G1 // 16

    def chunk(c, carry):
      pltpu.sync_copy(src_hbm.at[pl.ds(g0 + c * _CG, _CG)], src_v)
      pltpu.sync_copy(dst_hbm.at[pl.ds(g0 + c * _CG, _CG)], dst_v)
      for g in range(_NBUF):
        pltpu.async_copy(x_hbm.at[src_v.at[g]], rows_v.at[g], sems[g])
      for g in range(_CG):
        b = g % _NBUF
        pltpu.make_async_copy(x_hbm.at[src_v.at[g]], rows_v.at[b],
                              sems[b]).wait()
        pltpu.sync_copy(rows_v.at[b], agg_sh.at[dst_v.at[g]], add=True)
        if g + _NBUF < _CG:
          pltpu.async_copy(x_hbm.at[src_v.at[g + _NBUF]], rows_v.at[b],
                           sems[b])
        for i in range(vpr):
          d = dst_v[g, pl.ds(i * 16, 16)]
          plsc.addupdate_scatter(degacc_v, [d // 128, d % 128], ones)
      return carry

    lax.fori_loop(0, half_g // _CG, chunk, 0)
    pltpu.sync_copy(degacc_v, deg_sh.at[iota_v], add=True)
    plsc.subcore_barrier()

    @pl.when(cid == 0)
    def _():
      pltpu.sync_copy(agg_sh.at[pl.ds(r0, _RPT)], out0.at[pl.ds(r0, _RPT)])

      @pl.when(sid == 0)
      def _():
        pltpu.sync_copy(deg_sh, deg0)

    @pl.when(cid == 1)
    def _():
      pltpu.sync_copy(agg_sh.at[pl.ds(r0, _RPT)], out1.at[pl.ds(r0, _RPT)])

      @pl.when(sid == 0)
      def _():
        pltpu.sync_copy(deg_sh, deg1)

  return pl.kernel(body, out_type=out_type, mesh=_mesh,
                   scratch_types=scratch, compiler_params=_SC_PARAMS)


def _graph_norm_relu(h, g, be, a):
  mean = jnp.mean(h, axis=0, keepdims=True)
  sub = h - a * mean
  var = jnp.mean(sub * sub, axis=0, keepdims=True)
  return g * sub / jnp.sqrt(var + 1e-5) + be


def _tc_layer1(p0, p1, d0, d1, x, wlt, bl, wrt, g, be, a):
  out_shape = [jax.ShapeDtypeStruct((_N, 128), jnp.float32),
               jax.ShapeDtypeStruct((_N, 128), jnp.float32),
               jax.ShapeDtypeStruct((_N, 1), jnp.float32)]

  def body(p0_r, p1_r, d0_r, d1_r, x_r, wl_r, bl_r, wr_r, g_r, be_r, a_r,
           o0, o1, sc_r):
    scale = 1.0 / jnp.clip(d0_r[...] + d1_r[...], 1.0, None)
    agg = (p0_r[...][:_N] + p1_r[...][:_N]) * scale
    h = (jnp.dot(agg, wl_r[...], preferred_element_type=jnp.float32)
         + jnp.dot(x_r[...], wr_r[...], preferred_element_type=jnp.float32)
         + bl_r[...])
    gn = _graph_norm_relu(h, g_r[...], be_r[...], a_r[...])
    o0[...] = jnp.maximum(gn[:, :128], 0.0)
    o1[...] = jnp.maximum(gn[:, 128:], 0.0)
    sc_r[...] = scale

  return pl.pallas_call(body, out_shape=out_shape)(
      p0, p1, d0, d1, x, wlt, bl, wrt, g, be, a)


def _tc_layer(agg0, agg1, scale, x0, x1, wl0, wl1, bl, wr0, wr1, g, be, a,
              fcwt0=None, fcwt1=None):
  last = fcwt0 is not None
  if last:
    out_shape = [jax.ShapeDtypeStruct((_N, _NC), jnp.float32)]
  else:
    out_shape = [jax.ShapeDtypeStruct((_N, 128), jnp.float32),
                 jax.ShapeDtypeStruct((_N, 128), jnp.float32)]

  def body(agg0_r, agg1_r, sc_r, x0_r, x1_r, wl0_r, wl1_r, bl_r,
           wr0_r, wr1_r, g_r, be_r, a_r, *rest):
    scale = sc_r[...]
    a0 = agg0_r[...][:_N] * scale
    a1 = agg1_r[...][:_N] * scale
    xx0 = x0_r[...]
    xx1 = x1_r[...]
    h = (jnp.dot(a0, wl0_r[...], preferred_element_type=jnp.float32)
         + jnp.dot(a1, wl1_r[...], preferred_element_type=jnp.float32)
         + jnp.dot(xx0, wr0_r[...], preferred_element_type=jnp.float32)
         + jnp.dot(xx1, wr1_r[...], preferred_element_type=jnp.float32)
         + bl_r[...])
    gn = _graph_norm_relu(h, g_r[...], be_r[...], a_r[...])
    act0 = jnp.maximum(gn[:, :128] + xx0, 0.0)
    act1 = jnp.maximum(gn[:, 128:] + xx1, 0.0)
    if last:
      fw0_r, fw1_r, z_r = rest
      z_r[...] = (
          jnp.dot(act0, fw0_r[...], preferred_element_type=jnp.float32)
          + jnp.dot(act1, fw1_r[...], preferred_element_type=jnp.float32))
    else:
      rest[0][...] = act0
      rest[1][...] = act1

  args = [agg0, agg1, scale, x0, x1, wl0, wl1, bl, wr0, wr1, g, be, a]
  if last:
    args += [fcwt0, fcwt1]
  return pl.pallas_call(
      body, out_shape=out_shape,
      compiler_params=pltpu.CompilerParams(
          vmem_limit_bytes=100 * 1024 * 1024))(*args)


def _make_pool():
  out_type = jax.ShapeDtypeStruct((_NC * _ND,), jnp.float32)
  scratch = [
      pltpu.VMEM((_N * _NC,), jnp.float32),
      pltpu.VMEM((_DPT * _LPAD,), jnp.int32),
      pltpu.VMEM((_DPT * _LPAD,), jnp.float32),
      pltpu.VMEM((2 * 16,), jnp.float32),
      pltpu.VMEM((_NC * _DPT,), jnp.float32),
  ]

  def body(z_hbm, ids_hbm, w_hbm, fcb_hbm, out_hbm,
           z_v, ids_v, w_v, fcb_v, out_v):
    cid = lax.axis_index("c")
    sid = lax.axis_index("s")
    wid = sid * 2 + cid
    pltpu.sync_copy(z_hbm, z_v)
    pltpu.sync_copy(ids_hbm.at[pl.ds(wid * _DPT * _LPAD, _DPT * _LPAD)],
                    ids_v)
    pltpu.sync_copy(w_hbm.at[pl.ds(wid * _DPT * _LPAD, _DPT * _LPAD)], w_v)
    pltpu.sync_copy(fcb_hbm, fcb_v)
    fcb0 = fcb_v[pl.ds(0, 16)]
    fcb1 = fcb_v[pl.ds(16, 16)]
    for g in range(_DPT // 16):
      acc0 = jnp.zeros((16,), jnp.float32)
      acc1 = jnp.zeros((16,), jnp.float32)
      wacc = jnp.zeros((16,), jnp.float32)
      for l in range(_LPAD):
        off = l * _DPT + g * 16
        idx = ids_v[pl.ds(off, 16)]
        w = w_v[pl.ds(off, 16)]
        z0 = plsc.load_gather(z_v, [idx * 2])
        z1 = plsc.load_gather(z_v, [idx * 2 + 1])
        acc0 = acc0 + w * z0
        acc1 = acc1 + w * z1
        wacc = wacc + w
      inv = 1.0 / (wacc + 1e-8)
      out_v[pl.ds(g * 16, 16)] = acc0 * inv + fcb0
      out_v[pl.ds(_DPT + g * 16, 16)] = acc1 * inv + fcb1
    pltpu.sync_copy(out_v.at[pl.ds(0, _DPT)],
                    out_hbm.at[pl.ds(wid * _DPT, _DPT)])
    pltpu.sync_copy(out_v.at[pl.ds(_DPT, _DPT)],
                    out_hbm.at[pl.ds(_ND + wid * _DPT, _DPT)])

  return pl.kernel(body, out_type=out_type, mesh=_mesh,
                   scratch_types=scratch, compiler_params=_SC_PARAMS)


_segsum_l1 = _make_segsum_l1()
_segsum = _make_segsum()
_pool = _make_pool()


def kernel(x, edge_index, doc_word_ids, doc_weights,
           W1l, b1l, W1r, g1, be1, a1,
           W2l, b2l, W2r, g2, be2, a2,
           W3l, b3l, W3r, g3, be3, a3,
           W4l, b4l, W4r, g4, be4, a4,
           fcW, fcb):
  src = edge_index[0].astype(jnp.int32)
  dst = edge_index[1].astype(jnp.int32)
  pad_e = _EPAD - _E
  src_f = jnp.concatenate([src, jnp.zeros((pad_e,), jnp.int32)])
  dst_f = jnp.concatenate([dst, jnp.full((pad_e,), _N, jnp.int32)])
  src_p = src_f.reshape(_EPAD // _G, _G)
  dst_p = dst_f.reshape(_EPAD // _G, _G)
  src_p1 = src_f.reshape(_EPAD // _G1, _G1)
  dst_p1 = dst_f.reshape(_EPAD // _G1, _G1)
  z128 = jnp.zeros((_NPAD, 128), jnp.float32)
  iota = jnp.arange(_NPAD // 128, dtype=jnp.int32)

  def halves(w):
    wt = w.T
    h = wt.shape[0] // 2
    return wt[:h], wt[h:]

  p0, p1, deg0, deg1 = _segsum_l1(x, src_p1, dst_p1, z128, iota)
  d0 = deg0.reshape(-1, 1)[:_N]
  d1 = deg1.reshape(-1, 1)[:_N]
  h0, h1, scale = _tc_layer1(p0, p1, d0, d1, x, W1l.T, b1l[None],
                             W1r.T, g1[None], be1[None], a1[None])
  for Wl, bl, Wr, g, be, a in ((W2l, b2l, W2r, g2, be2, a2),
                               (W3l, b3l, W3r, g3, be3, a3)):
    agg0, agg1 = _segsum(h0, h1, src_p, dst_p, z128)
    wl0, wl1 = halves(Wl)
    wr0, wr1 = halves(Wr)
    h0, h1 = _tc_layer(agg0, agg1, scale, h0, h1, wl0, wl1, bl[None],
                       wr0, wr1, g[None], be[None], a[None])
  agg0, agg1 = _segsum(h0, h1, src_p, dst_p, z128)
  wl0, wl1 = halves(W4l)
  wr0, wr1 = halves(W4r)
  fw0, fw1 = halves(fcW)
  z = _tc_layer(agg0, agg1, scale, h0, h1, wl0, wl1, b4l[None],
                wr0, wr1, g4[None], be4[None], a4[None],
                fcwt0=fw0, fcwt1=fw1)[0]

  def tileize(arr):
    p = jnp.pad(arr, ((0, 0), (0, _LPAD - _L)))
    return p.T.reshape(_LPAD, 32, _DPT).transpose(1, 0, 2).reshape(-1)

  ids_f = tileize(doc_word_ids.astype(jnp.int32))
  w_f = tileize(doc_weights)
  out_f = _pool(z.reshape(-1), ids_f, w_f, jnp.repeat(fcb, 16))
  return out_f.reshape(_NC, _ND).T


# speedup vs baseline: 1.1973x; 1.0546x over previous
import functools

import jax
import jax.numpy as jnp
from jax import lax
from jax.experimental import pallas as pl
from jax.experimental.pallas import tpu as pltpu
from jax.experimental.pallas import tpu_sc as plsc

_N = 10000
_E = 320000
_NC = 2
_ND = 1024
_L = 50

_G = 80
_TILES = 16
_GPT = 256
_EPAD = _TILES * _G * _GPT
_G1 = 64
_GPT1 = _EPAD // (_TILES * _G1)
_NPAD = 10240
_RPT = _NPAD // _TILES
_LPAD = 64
_DPT = _ND // 32
_CG = 16
_NBUF = 4

_mesh = plsc.VectorSubcoreMesh(core_axis_name="c", subcore_axis_name="s")
_SC_PARAMS = pltpu.CompilerParams(needs_layout_passes=False)


def _make_segsum():
  out_type = [jax.ShapeDtypeStruct((_NPAD, 128), jnp.float32),
              jax.ShapeDtypeStruct((_NPAD, 128), jnp.float32)]
  nchunks = _GPT // _CG
  scratch = [
      pltpu.VMEM((2, _CG, _G), jnp.int32),
      pltpu.VMEM((2, _CG, _G), jnp.int32),
      pltpu.VMEM((_NBUF, _G, 128), jnp.float32),
      pltpu.VMEM_SHARED((_NPAD, 128), jnp.float32),
      [pltpu.SemaphoreType.DMA] * _NBUF,
      pltpu.SemaphoreType.DMA,
      pltpu.SemaphoreType.DMA,
  ]

  def body(x0_hbm, x1_hbm, src_hbm, dst_hbm, zw_hbm,
           out0, out1, src_v, dst_v, rows_v, agg_sh, sems, isem0, isem1):
    cid = lax.axis_index("c")
    sid = lax.axis_index("s")
    r0 = sid * _RPT
    g0 = sid * _GPT
    pltpu.sync_copy(zw_hbm.at[pl.ds(r0, _RPT)], agg_sh.at[pl.ds(r0, _RPT)])
    pltpu.sync_copy(src_hbm.at[pl.ds(g0, _CG)], src_v.at[0])
    pltpu.sync_copy(dst_hbm.at[pl.ds(g0, _CG)], dst_v.at[0])
    plsc.subcore_barrier()

    def run(x_ref):
      for g in range(_NBUF):
        pltpu.async_copy(x_ref.at[src_v.at[0].at[g]], rows_v.at[g], sems[g])

      def chunk(c, carry):
        cur = c % 2
        nxt = (c + 1) % 2
        last = c + 1 >= nchunks

        for g in range(_CG):
          b = g % _NBUF
          pltpu.make_async_copy(x_ref.at[src_v.at[cur].at[g]],
                                rows_v.at[b], sems[b]).wait()
          pltpu.sync_copy(rows_v.at[b], agg_sh.at[dst_v.at[cur].at[g]],
                          add=True)
          if g == _NBUF - 1:
            @pl.when(~last)
            def _():
              pltpu.async_copy(src_hbm.at[pl.ds(g0 + (c + 1) * _CG, _CG)],
                               src_v.at[nxt], isem0)
              pltpu.async_copy(dst_hbm.at[pl.ds(g0 + (c + 1) * _CG, _CG)],
                               dst_v.at[nxt], isem1)
          if g + _NBUF < _CG:
            pltpu.async_copy(x_ref.at[src_v.at[cur].at[g + _NBUF]],
                             rows_v.at[b], sems[b])
          else:
            if g == _CG - _NBUF:
              @pl.when(~last)
              def _():
                pltpu.make_async_copy(
                    src_hbm.at[pl.ds(g0 + (c + 1) * _CG, _CG)],
                    src_v.at[nxt], isem0).wait()
                pltpu.make_async_copy(
                    dst_hbm.at[pl.ds(g0 + (c + 1) * _CG, _CG)],
                    dst_v.at[nxt], isem1).wait()

            @pl.when(~last)
            def _():
              pltpu.async_copy(
                  x_ref.at[src_v.at[nxt].at[g + _NBUF - _CG]],
                  rows_v.at[b], sems[b])
        return carry
      lax.fori_loop(0, nchunks, chunk, 0)

    @pl.when(cid == 0)
    def _():
      run(x0_hbm)

    @pl.when(cid == 1)
    def _():
      run(x1_hbm)

    plsc.subcore_barrier()

    @pl.when(cid == 0)
    def _():
      pltpu.sync_copy(agg_sh.at[pl.ds(r0, _RPT)], out0.at[pl.ds(r0, _RPT)])

    @pl.when(cid == 1)
    def _():
      pltpu.sync_copy(agg_sh.at[pl.ds(r0, _RPT)], out1.at[pl.ds(r0, _RPT)])

  return pl.kernel(body, out_type=out_type, mesh=_mesh,
                   scratch_types=scratch, compiler_params=_SC_PARAMS)


def _make_segsum_l1():
  half_g = _GPT1 // 2
  drows = _NPAD // 128
  out_type = [jax.ShapeDtypeStruct((_NPAD, 128), jnp.float32),
              jax.ShapeDtypeStruct((_NPAD, 128), jnp.float32),
              jax.ShapeDtypeStruct((drows, 128), jnp.float32),
              jax.ShapeDtypeStruct((drows, 128), jnp.float32)]
  scratch = [
      pltpu.VMEM((_CG, _G1), jnp.int32),
      pltpu.VMEM((_CG, _G1), jnp.int32),
      pltpu.VMEM((_NBUF, _G1, 128), jnp.float32),
      pltpu.VMEM((drows, 128), jnp.float32),
      pltpu.VMEM((drows,), jnp.int32),
      pltpu.VMEM_SHARED((_NPAD, 128), jnp.float32),
      pltpu.VMEM_SHARED((drows, 128), jnp.float32),
      [pltpu.SemaphoreType.DMA] * _NBUF,
  ]

  def body(x_hbm, src_hbm, dst_hbm, zw_hbm, iota_hbm,
           out0, out1, deg0, deg1,
           src_v, dst_v, rows_v, degacc_v, iota_v,
           agg_sh, deg_sh, sems):
    cid = lax.axis_index("c")
    sid = lax.axis_index("s")
    r0 = sid * _RPT
    g0 = cid * (half_g * _TILES) + sid * half_g
    pltpu.sync_copy(zw_hbm.at[pl.ds(r0, _RPT)], agg_sh.at[pl.ds(r0, _RPT)])
    pltpu.sync_copy(zw_hbm.at[pl.ds(0, drows)], degacc_v)
    pltpu.sync_copy(iota_hbm, iota_v)

    @pl.when(sid == 0)
    def _():
      pltpu.sync_copy(zw_hbm.at[pl.ds(0, drows)], deg_sh)

    plsc.subcore_barrier()

    ones = jnp.ones((16,), jnp.float32)

    vpr = _G1 // 16

    def chunk(c, carry):
      pltpu.sync_copy(src_hbm.at[pl.ds(g0 + c * _CG, _CG)], src_v)
      pltpu.sync_copy(dst_hbm.at[pl.ds(g0 + c * _CG, _CG)], dst_v)
      for g in range(_NBUF):
        pltpu.async_copy(x_hbm.at[src_v.at[g]], rows_v.at[g], sems[g])
      for g in range(_CG):
        b = g % _NBUF
        pltpu.make_async_copy(x_hbm.at[src_v.at[g]], rows_v.at[b],
                              sems[b]).wait()
        pltpu.sync_copy(rows_v.at[b], agg_sh.at[dst_v.at[g]], add=True)
        if g + _NBUF < _CG:
          pltpu.async_copy(x_hbm.at[src_v.at[g + _NBUF]], rows_v.at[b],
                           sems[b])
        for i in range(vpr):
          d = dst_v[g, pl.ds(i * 16, 16)]
          plsc.addupdate_scatter(degacc_v, [d // 128, d % 128], ones)
      return carry

    lax.fori_loop(0, half_g // _CG, chunk, 0)
    pltpu.sync_copy(degacc_v, deg_sh.at[iota_v], add=True)
    plsc.subcore_barrier()

    @pl.when(cid == 0)
    def _():
      pltpu.sync_copy(agg_sh.at[pl.ds(r0, _RPT)], out0.at[pl.ds(r0, _RPT)])

      @pl.when(sid == 0)
      def _():
        pltpu.sync_copy(deg_sh, deg0)

    @pl.when(cid == 1)
    def _():
      pltpu.sync_copy(agg_sh.at[pl.ds(r0, _RPT)], out1.at[pl.ds(r0, _RPT)])

      @pl.when(sid == 0)
      def _():
        pltpu.sync_copy(deg_sh, deg1)

  return pl.kernel(body, out_type=out_type, mesh=_mesh,
                   scratch_types=scratch, compiler_params=_SC_PARAMS)


def _graph_norm_relu(h, g, be, a):
  mean = jnp.mean(h, axis=0, keepdims=True)
  sub = h - a * mean
  var = jnp.mean(sub * sub, axis=0, keepdims=True)
  return g * sub / jnp.sqrt(var + 1e-5) + be


def _tc_layer1(p0, p1, d0, d1, x, wlt, bl, wrt, g, be, a):
  out_shape = [jax.ShapeDtypeStruct((_N, 128), jnp.float32),
               jax.ShapeDtypeStruct((_N, 128), jnp.float32),
               jax.ShapeDtypeStruct((_N, 1), jnp.float32)]

  def body(p0_r, p1_r, d0_r, d1_r, x_r, wl_r, bl_r, wr_r, g_r, be_r, a_r,
           o0, o1, sc_r):
    scale = 1.0 / jnp.clip(d0_r[...] + d1_r[...], 1.0, None)
    agg = (p0_r[...][:_N] + p1_r[...][:_N]) * scale
    h = (jnp.dot(agg, wl_r[...], preferred_element_type=jnp.float32)
         + jnp.dot(x_r[...], wr_r[...], preferred_element_type=jnp.float32)
         + bl_r[...])
    gn = _graph_norm_relu(h, g_r[...], be_r[...], a_r[...])
    o0[...] = jnp.maximum(gn[:, :128], 0.0)
    o1[...] = jnp.maximum(gn[:, 128:], 0.0)
    sc_r[...] = scale

  return pl.pallas_call(body, out_shape=out_shape)(
      p0, p1, d0, d1, x, wlt, bl, wrt, g, be, a)


def _tc_layer(agg0, agg1, scale, x0, x1, wl0, wl1, bl, wr0, wr1, g, be, a,
              fcwt0=None, fcwt1=None):
  last = fcwt0 is not None
  if last:
    out_shape = [jax.ShapeDtypeStruct((_N, _NC), jnp.float32)]
  else:
    out_shape = [jax.ShapeDtypeStruct((_N, 128), jnp.float32),
                 jax.ShapeDtypeStruct((_N, 128), jnp.float32)]

  def body(agg0_r, agg1_r, sc_r, x0_r, x1_r, wl0_r, wl1_r, bl_r,
           wr0_r, wr1_r, g_r, be_r, a_r, *rest):
    scale = sc_r[...]
    a0 = agg0_r[...][:_N] * scale
    a1 = agg1_r[...][:_N] * scale
    xx0 = x0_r[...]
    xx1 = x1_r[...]
    h = (jnp.dot(a0, wl0_r[...], preferred_element_type=jnp.float32)
         + jnp.dot(a1, wl1_r[...], preferred_element_type=jnp.float32)
         + jnp.dot(xx0, wr0_r[...], preferred_element_type=jnp.float32)
         + jnp.dot(xx1, wr1_r[...], preferred_element_type=jnp.float32)
         + bl_r[...])
    gn = _graph_norm_relu(h, g_r[...], be_r[...], a_r[...])
    act0 = jnp.maximum(gn[:, :128] + xx0, 0.0)
    act1 = jnp.maximum(gn[:, 128:] + xx1, 0.0)
    if last:
      fw0_r, fw1_r, z_r = rest
      z_r[...] = (
          jnp.dot(act0, fw0_r[...], preferred_element_type=jnp.float32)
          + jnp.dot(act1, fw1_r[...], preferred_element_type=jnp.float32))
    else:
      rest[0][...] = act0
      rest[1][...] = act1

  args = [agg0, agg1, scale, x0, x1, wl0, wl1, bl, wr0, wr1, g, be, a]
  if last:
    args += [fcwt0, fcwt1]
  return pl.pallas_call(
      body, out_shape=out_shape,
      compiler_params=pltpu.CompilerParams(
          vmem_limit_bytes=100 * 1024 * 1024))(*args)


def _make_pool():
  out_type = jax.ShapeDtypeStruct((_NC * _ND,), jnp.float32)
  scratch = [
      pltpu.VMEM((_N * _NC,), jnp.float32),
      pltpu.VMEM((_DPT * _LPAD,), jnp.int32),
      pltpu.VMEM((_DPT * _LPAD,), jnp.float32),
      pltpu.VMEM((2 * 16,), jnp.float32),
      pltpu.VMEM((_NC * _DPT,), jnp.float32),
  ]

  def body(z_hbm, ids_hbm, w_hbm, fcb_hbm, out_hbm,
           z_v, ids_v, w_v, fcb_v, out_v):
    cid = lax.axis_index("c")
    sid = lax.axis_index("s")
    wid = sid * 2 + cid
    pltpu.sync_copy(z_hbm, z_v)
    pltpu.sync_copy(ids_hbm.at[pl.ds(wid * _DPT * _LPAD, _DPT * _LPAD)],
                    ids_v)
    pltpu.sync_copy(w_hbm.at[pl.ds(wid * _DPT * _LPAD, _DPT * _LPAD)], w_v)
    pltpu.sync_copy(fcb_hbm, fcb_v)
    fcb0 = fcb_v[pl.ds(0, 16)]
    fcb1 = fcb_v[pl.ds(16, 16)]
    for g in range(_DPT // 16):
      acc0 = jnp.zeros((16,), jnp.float32)
      acc1 = jnp.zeros((16,), jnp.float32)
      wacc = jnp.zeros((16,), jnp.float32)
      for l in range(_LPAD):
        off = l * _DPT + g * 16
        idx = ids_v[pl.ds(off, 16)]
        w = w_v[pl.ds(off, 16)]
        z0 = plsc.load_gather(z_v, [idx * 2])
        z1 = plsc.load_gather(z_v, [idx * 2 + 1])
        acc0 = acc0 + w * z0
        acc1 = acc1 + w * z1
        wacc = wacc + w
      inv = 1.0 / (wacc + 1e-8)
      out_v[pl.ds(g * 16, 16)] = acc0 * inv + fcb0
      out_v[pl.ds(_DPT + g * 16, 16)] = acc1 * inv + fcb1
    pltpu.sync_copy(out_v.at[pl.ds(0, _DPT)],
                    out_hbm.at[pl.ds(wid * _DPT, _DPT)])
    pltpu.sync_copy(out_v.at[pl.ds(_DPT, _DPT)],
                    out_hbm.at[pl.ds(_ND + wid * _DPT, _DPT)])

  return pl.kernel(body, out_type=out_type, mesh=_mesh,
                   scratch_types=scratch, compiler_params=_SC_PARAMS)


_segsum_l1 = _make_segsum_l1()
_segsum = _make_segsum()
_pool = _make_pool()


def kernel(x, edge_index, doc_word_ids, doc_weights,
           W1l, b1l, W1r, g1, be1, a1,
           W2l, b2l, W2r, g2, be2, a2,
           W3l, b3l, W3r, g3, be3, a3,
           W4l, b4l, W4r, g4, be4, a4,
           fcW, fcb):
  src = edge_index[0].astype(jnp.int32)
  dst = edge_index[1].astype(jnp.int32)
  pad_e = _EPAD - _E
  src_f = jnp.concatenate([src, jnp.zeros((pad_e,), jnp.int32)])
  dst_f = jnp.concatenate([dst, jnp.full((pad_e,), _N, jnp.int32)])
  src_p = src_f.reshape(_EPAD // _G, _G)
  dst_p = dst_f.reshape(_EPAD // _G, _G)
  src_p1 = src_f.reshape(_EPAD // _G1, _G1)
  dst_p1 = dst_f.reshape(_EPAD // _G1, _G1)
  z128 = jnp.zeros((_NPAD, 128), jnp.float32)
  iota = jnp.arange(_NPAD // 128, dtype=jnp.int32)

  def halves(w):
    wt = w.T
    h = wt.shape[0] // 2
    return wt[:h], wt[h:]

  p0, p1, deg0, deg1 = _segsum_l1(x, src_p1, dst_p1, z128, iota)
  d0 = deg0.reshape(-1, 1)[:_N]
  d1 = deg1.reshape(-1, 1)[:_N]
  h0, h1, scale = _tc_layer1(p0, p1, d0, d1, x, W1l.T, b1l[None],
                             W1r.T, g1[None], be1[None], a1[None])
  for Wl, bl, Wr, g, be, a in ((W2l, b2l, W2r, g2, be2, a2),
                               (W3l, b3l, W3r, g3, be3, a3)):
    agg0, agg1 = _segsum(h0, h1, src_p, dst_p, z128)
    wl0, wl1 = halves(Wl)
    wr0, wr1 = halves(Wr)
    h0, h1 = _tc_layer(agg0, agg1, scale, h0, h1, wl0, wl1, bl[None],
                       wr0, wr1, g[None], be[None], a[None])
  agg0, agg1 = _segsum(h0, h1, src_p, dst_p, z128)
  wl0, wl1 = halves(W4l)
  wr0, wr1 = halves(W4r)
  fw0, fw1 = halves(fcW)
  z = _tc_layer(agg0, agg1, scale, h0, h1, wl0, wl1, b4l[None],
                wr0, wr1, g4[None], be4[None], a4[None],
                fcwt0=fw0, fcwt1=fw1)[0]

  def tileize(arr):
    p = jnp.pad(arr, ((0, 0), (0, _LPAD - _L)))
    return p.T.reshape(_LPAD, 32, _DPT).transpose(1, 0, 2).reshape(-1)

  ids_f = tileize(doc_word_ids.astype(jnp.int32))
  w_f = tileize(doc_weights)
  out_f = _pool(z.reshape(-1), ids_f, w_f, jnp.repeat(fcb, 16))
  return out_f.reshape(_NC, _ND).T


# speedup vs baseline: 1.1978x; 1.0005x over previous
import functools

import jax
import jax.numpy as jnp
from jax import lax
from jax.experimental import pallas as pl
from jax.experimental.pallas import tpu as pltpu
from jax.experimental.pallas import tpu_sc as plsc

_N = 10000
_E = 320000
_NC = 2
_ND = 1024
_L = 50

_G = 80
_TILES = 16
_GPT = 256
_EPAD = _TILES * _G * _GPT
_G1 = 64
_GPT1 = _EPAD // (_TILES * _G1)
_NPAD = 10240
_RPT = _NPAD // _TILES
_LPAD = 64
_DPT = _ND // 32
_CG = 16
_NBUF = 4

_mesh = plsc.VectorSubcoreMesh(core_axis_name="c", subcore_axis_name="s")
_SC_PARAMS = pltpu.CompilerParams(needs_layout_passes=False)


def _make_segsum():
  out_type = [jax.ShapeDtypeStruct((_NPAD, 128), jnp.float32),
              jax.ShapeDtypeStruct((_NPAD, 128), jnp.float32)]
  nchunks = _GPT // _CG
  scratch = [
      pltpu.VMEM((2, _CG, _G), jnp.int32),
      pltpu.VMEM((2, _CG, _G), jnp.int32),
      pltpu.VMEM((_NBUF, _G, 128), jnp.float32),
      pltpu.VMEM_SHARED((_NPAD, 128), jnp.float32),
      [pltpu.SemaphoreType.DMA] * _NBUF,
      pltpu.SemaphoreType.DMA,
      pltpu.SemaphoreType.DMA,
  ]

  def body(x0_hbm, x1_hbm, src_hbm, dst_hbm, zw_hbm,
           out0, out1, src_v, dst_v, rows_v, agg_sh, sems, isem0, isem1):
    cid = lax.axis_index("c")
    sid = lax.axis_index("s")
    r0 = sid * _RPT
    g0 = sid * _GPT
    pltpu.sync_copy(zw_hbm.at[pl.ds(r0, _RPT)], agg_sh.at[pl.ds(r0, _RPT)])
    pltpu.sync_copy(src_hbm.at[pl.ds(g0, _CG)], src_v.at[0])
    pltpu.sync_copy(dst_hbm.at[pl.ds(g0, _CG)], dst_v.at[0])
    plsc.subcore_barrier()

    def run(x_ref):
      for g in range(_NBUF):
        pltpu.async_copy(x_ref.at[src_v.at[0].at[g]], rows_v.at[g], sems[g])

      def chunk(c, carry):
        cur = c % 2
        nxt = (c + 1) % 2
        last = c + 1 >= nchunks

        for g in range(_CG):
          b = g % _NBUF
          pltpu.make_async_copy(x_ref.at[src_v.at[cur].at[g]],
                                rows_v.at[b], sems[b]).wait()
          pltpu.sync_copy(rows_v.at[b], agg_sh.at[dst_v.at[cur].at[g]],
                          add=True)
          if g == _NBUF - 1:
            @pl.when(~last)
            def _():
              pltpu.async_copy(src_hbm.at[pl.ds(g0 + (c + 1) * _CG, _CG)],
                               src_v.at[nxt], isem0)
              pltpu.async_copy(dst_hbm.at[pl.ds(g0 + (c + 1) * _CG, _CG)],
                               dst_v.at[nxt], isem1)
          if g + _NBUF < _CG:
            pltpu.async_copy(x_ref.at[src_v.at[cur].at[g + _NBUF]],
                             rows_v.at[b], sems[b])
          else:
            if g == _CG - _NBUF:
              @pl.when(~last)
              def _():
                pltpu.make_async_copy(
                    src_hbm.at[pl.ds(g0 + (c + 1) * _CG, _CG)],
                    src_v.at[nxt], isem0).wait()
                pltpu.make_async_copy(
                    dst_hbm.at[pl.ds(g0 + (c + 1) * _CG, _CG)],
                    dst_v.at[nxt], isem1).wait()

            @pl.when(~last)
            def _():
              pltpu.async_copy(
                  x_ref.at[src_v.at[nxt].at[g + _NBUF - _CG]],
                  rows_v.at[b], sems[b])
        return carry
      lax.fori_loop(0, nchunks, chunk, 0)

    @pl.when(cid == 0)
    def _():
      run(x0_hbm)

    @pl.when(cid == 1)
    def _():
      run(x1_hbm)

    plsc.subcore_barrier()

    @pl.when(cid == 0)
    def _():
      pltpu.sync_copy(agg_sh.at[pl.ds(r0, _RPT)], out0.at[pl.ds(r0, _RPT)])

    @pl.when(cid == 1)
    def _():
      pltpu.sync_copy(agg_sh.at[pl.ds(r0, _RPT)], out1.at[pl.ds(r0, _RPT)])

  return pl.kernel(body, out_type=out_type, mesh=_mesh,
                   scratch_types=scratch, compiler_params=_SC_PARAMS)


def _make_segsum_l1():
  half_g = _GPT1 // 2
  drows = _NPAD // 128
  out_type = [jax.ShapeDtypeStruct((_NPAD, 128), jnp.float32),
              jax.ShapeDtypeStruct((_NPAD, 128), jnp.float32),
              jax.ShapeDtypeStruct((drows, 128), jnp.float32),
              jax.ShapeDtypeStruct((drows, 128), jnp.float32)]
  nchunks = half_g // _CG
  scratch = [
      pltpu.VMEM((_CG, _G1), jnp.int32),
      pltpu.VMEM((_CG, _G1), jnp.int32),
      pltpu.VMEM((_NBUF, _G1, 128), jnp.float32),
      pltpu.VMEM((drows, 128), jnp.float32),
      pltpu.VMEM((drows,), jnp.int32),
      pltpu.VMEM_SHARED((_NPAD, 128), jnp.float32),
      pltpu.VMEM_SHARED((drows, 128), jnp.float32),
      [pltpu.SemaphoreType.DMA] * _NBUF,
  ]

  def body(x_hbm, src_hbm, dst_hbm, zw_hbm, iota_hbm,
           out0, out1, deg0, deg1,
           src_v, dst_v, rows_v, degacc_v, iota_v,
           agg_sh, deg_sh, sems):
    cid = lax.axis_index("c")
    sid = lax.axis_index("s")
    r0 = sid * _RPT
    g0 = cid * (half_g * _TILES) + sid * half_g
    pltpu.sync_copy(zw_hbm.at[pl.ds(r0, _RPT)], agg_sh.at[pl.ds(r0, _RPT)])
    pltpu.sync_copy(zw_hbm.at[pl.ds(0, drows)], degacc_v)
    pltpu.sync_copy(iota_hbm, iota_v)

    @pl.when(sid == 0)
    def _():
      pltpu.sync_copy(zw_hbm.at[pl.ds(0, drows)], deg_sh)

    plsc.subcore_barrier()

    ones = jnp.ones((16,), jnp.float32)

    vpr = _G1 // 16

    def chunk(c, carry):
      pltpu.sync_copy(src_hbm.at[pl.ds(g0 + c * _CG, _CG)], src_v)
      pltpu.sync_copy(dst_hbm.at[pl.ds(g0 + c * _CG, _CG)], dst_v)
      for g in range(_NBUF):
        pltpu.async_copy(x_hbm.at[src_v.at[g]], rows_v.at[g], sems[g])
      for g in range(_CG):
        b = g % _NBUF
        pltpu.make_async_copy(x_hbm.at[src_v.at[g]], rows_v.at[b],
                              sems[b]).wait()
        pltpu.sync_copy(rows_v.at[b], agg_sh.at[dst_v.at[g]], add=True)
        if g + _NBUF < _CG:
          pltpu.async_copy(x_hbm.at[src_v.at[g + _NBUF]], rows_v.at[b],
                           sems[b])
        for i in range(vpr):
          d = dst_v[g, pl.ds(i * 16, 16)]
          plsc.addupdate_scatter(degacc_v, [d // 128, d % 128], ones)
      return carry

    lax.fori_loop(0, nchunks, chunk, 0)
    pltpu.sync_copy(degacc_v, deg_sh.at[iota_v], add=True)
    plsc.subcore_barrier()

    @pl.when(cid == 0)
    def _():
      pltpu.sync_copy(agg_sh.at[pl.ds(r0, _RPT)], out0.at[pl.ds(r0, _RPT)])

      @pl.when(sid == 0)
      def _():
        pltpu.sync_copy(deg_sh, deg0)

    @pl.when(cid == 1)
    def _():
      pltpu.sync_copy(agg_sh.at[pl.ds(r0, _RPT)], out1.at[pl.ds(r0, _RPT)])

      @pl.when(sid == 0)
      def _():
        pltpu.sync_copy(deg_sh, deg1)

  return pl.kernel(body, out_type=out_type, mesh=_mesh,
                   scratch_types=scratch, compiler_params=_SC_PARAMS)


def _graph_norm_relu(h, g, be, a):
  mean = jnp.mean(h, axis=0, keepdims=True)
  sub = h - a * mean
  var = jnp.mean(sub * sub, axis=0, keepdims=True)
  return g * sub / jnp.sqrt(var + 1e-5) + be


def _tc_layer1(p0, p1, d0, d1, x, wlt, bl, wrt, g, be, a):
  out_shape = [jax.ShapeDtypeStruct((_N, 128), jnp.float32),
               jax.ShapeDtypeStruct((_N, 128), jnp.float32),
               jax.ShapeDtypeStruct((_N, 1), jnp.float32)]

  def body(p0_r, p1_r, d0_r, d1_r, x_r, wl_r, bl_r, wr_r, g_r, be_r, a_r,
           o0, o1, sc_r):
    scale = 1.0 / jnp.clip(d0_r[...] + d1_r[...], 1.0, None)
    agg = (p0_r[...][:_N] + p1_r[...][:_N]) * scale
    h = (jnp.dot(agg, wl_r[...], preferred_element_type=jnp.float32)
         + jnp.dot(x_r[...], wr_r[...], preferred_element_type=jnp.float32)
         + bl_r[...])
    gn = _graph_norm_relu(h, g_r[...], be_r[...], a_r[...])
    o0[...] = jnp.maximum(gn[:, :128], 0.0)
    o1[...] = jnp.maximum(gn[:, 128:], 0.0)
    sc_r[...] = scale

  return pl.pallas_call(body, out_shape=out_shape)(
      p0, p1, d0, d1, x, wlt, bl, wrt, g, be, a)


def _tc_layer(agg0, agg1, scale, x0, x1, wl0, wl1, bl, wr0, wr1, g, be, a,
              fcwt0=None, fcwt1=None):
  last = fcwt0 is not None
  if last:
    out_shape = [jax.ShapeDtypeStruct((_N, _NC), jnp.float32)]
  else:
    out_shape = [jax.ShapeDtypeStruct((_N, 128), jnp.float32),
                 jax.ShapeDtypeStruct((_N, 128), jnp.float32)]

  def body(agg0_r, agg1_r, sc_r, x0_r, x1_r, wl0_r, wl1_r, bl_r,
           wr0_r, wr1_r, g_r, be_r, a_r, *rest):
    scale = sc_r[...]
    a0 = agg0_r[...][:_N] * scale
    a1 = agg1_r[...][:_N] * scale
    xx0 = x0_r[...]
    xx1 = x1_r[...]
    h = (jnp.dot(a0, wl0_r[...], preferred_element_type=jnp.float32)
         + jnp.dot(a1, wl1_r[...], preferred_element_type=jnp.float32)
         + jnp.dot(xx0, wr0_r[...], preferred_element_type=jnp.float32)
         + jnp.dot(xx1, wr1_r[...], preferred_element_type=jnp.float32)
         + bl_r[...])
    gn = _graph_norm_relu(h, g_r[...], be_r[...], a_r[...])
    act0 = jnp.maximum(gn[:, :128] + xx0, 0.0)
    act1 = jnp.maximum(gn[:, 128:] + xx1, 0.0)
    if last:
      fw0_r, fw1_r, z_r = rest
      z_r[...] = (
          jnp.dot(act0, fw0_r[...], preferred_element_type=jnp.float32)
          + jnp.dot(act1, fw1_r[...], preferred_element_type=jnp.float32))
    else:
      rest[0][...] = act0
      rest[1][...] = act1

  args = [agg0, agg1, scale, x0, x1, wl0, wl1, bl, wr0, wr1, g, be, a]
  if last:
    args += [fcwt0, fcwt1]
  return pl.pallas_call(
      body, out_shape=out_shape,
      compiler_params=pltpu.CompilerParams(
          vmem_limit_bytes=100 * 1024 * 1024))(*args)


def _make_pool():
  out_type = jax.ShapeDtypeStruct((_NC * _ND,), jnp.float32)
  scratch = [
      pltpu.VMEM((_N * _NC,), jnp.float32),
      pltpu.VMEM((_DPT * _LPAD,), jnp.int32),
      pltpu.VMEM((_DPT * _LPAD,), jnp.float32),
      pltpu.VMEM((2 * 16,), jnp.float32),
      pltpu.VMEM((_NC * _DPT,), jnp.float32),
  ]

  def body(z_hbm, ids_hbm, w_hbm, fcb_hbm, out_hbm,
           z_v, ids_v, w_v, fcb_v, out_v):
    cid = lax.axis_index("c")
    sid = lax.axis_index("s")
    wid = sid * 2 + cid
    pltpu.sync_copy(z_hbm, z_v)
    pltpu.sync_copy(ids_hbm.at[pl.ds(wid * _DPT * _LPAD, _DPT * _LPAD)],
                    ids_v)
    pltpu.sync_copy(w_hbm.at[pl.ds(wid * _DPT * _LPAD, _DPT * _LPAD)], w_v)
    pltpu.sync_copy(fcb_hbm, fcb_v)
    fcb0 = fcb_v[pl.ds(0, 16)]
    fcb1 = fcb_v[pl.ds(16, 16)]
    for g in range(_DPT // 16):
      acc0 = jnp.zeros((16,), jnp.float32)
      acc1 = jnp.zeros((16,), jnp.float32)
      wacc = jnp.zeros((16,), jnp.float32)
      for l in range(_LPAD):
        off = l * _DPT + g * 16
        idx = ids_v[pl.ds(off, 16)]
        w = w_v[pl.ds(off, 16)]
        z0 = plsc.load_gather(z_v, [idx * 2])
        z1 = plsc.load_gather(z_v, [idx * 2 + 1])
        acc0 = acc0 + w * z0
        acc1 = acc1 + w * z1
        wacc = wacc + w
      inv = 1.0 / (wacc + 1e-8)
      out_v[pl.ds(g * 16, 16)] = acc0 * inv + fcb0
      out_v[pl.ds(_DPT + g * 16, 16)] = acc1 * inv + fcb1
    pltpu.sync_copy(out_v.at[pl.ds(0, _DPT)],
                    out_hbm.at[pl.ds(wid * _DPT, _DPT)])
    pltpu.sync_copy(out_v.at[pl.ds(_DPT, _DPT)],
                    out_hbm.at[pl.ds(_ND + wid * _DPT, _DPT)])

  return pl.kernel(body, out_type=out_type, mesh=_mesh,
                   scratch_types=scratch, compiler_params=_SC_PARAMS)


_segsum_l1 = _make_segsum_l1()
_segsum = _make_segsum()
_pool = _make_pool()


def kernel(x, edge_index, doc_word_ids, doc_weights,
           W1l, b1l, W1r, g1, be1, a1,
           W2l, b2l, W2r, g2, be2, a2,
           W3l, b3l, W3r, g3, be3, a3,
           W4l, b4l, W4r, g4, be4, a4,
           fcW, fcb):
  src = edge_index[0].astype(jnp.int32)
  dst = edge_index[1].astype(jnp.int32)
  pad_e = _EPAD - _E
  src_f = jnp.concatenate([src, jnp.zeros((pad_e,), jnp.int32)])
  dst_f = jnp.concatenate([dst, jnp.full((pad_e,), _N, jnp.int32)])
  src_p = src_f.reshape(_EPAD // _G, _G)
  dst_p = dst_f.reshape(_EPAD // _G, _G)
  src_p1 = src_f.reshape(_EPAD // _G1, _G1)
  dst_p1 = dst_f.reshape(_EPAD // _G1, _G1)
  z128 = jnp.zeros((_NPAD, 128), jnp.float32)
  iota = jnp.arange(_NPAD // 128, dtype=jnp.int32)

  def halves(w):
    wt = w.T
    h = wt.shape[0] // 2
    return wt[:h], wt[h:]

  p0, p1, deg0, deg1 = _segsum_l1(x, src_p1, dst_p1, z128, iota)
  d0 = deg0.reshape(-1, 1)[:_N]
  d1 = deg1.reshape(-1, 1)[:_N]
  h0, h1, scale = _tc_layer1(p0, p1, d0, d1, x, W1l.T, b1l[None],
                             W1r.T, g1[None], be1[None], a1[None])
  for Wl, bl, Wr, g, be, a in ((W2l, b2l, W2r, g2, be2, a2),
                               (W3l, b3l, W3r, g3, be3, a3)):
    agg0, agg1 = _segsum(h0, h1, src_p, dst_p, z128)
    wl0, wl1 = halves(Wl)
    wr0, wr1 = halves(Wr)
    h0, h1 = _tc_layer(agg0, agg1, scale, h0, h1, wl0, wl1, bl[None],
                       wr0, wr1, g[None], be[None], a[None])
  agg0, agg1 = _segsum(h0, h1, src_p, dst_p, z128)
  wl0, wl1 = halves(W4l)
  wr0, wr1 = halves(W4r)
  fw0, fw1 = halves(fcW)
  z = _tc_layer(agg0, agg1, scale, h0, h1, wl0, wl1, b4l[None],
                wr0, wr1, g4[None], be4[None], a4[None],
                fcwt0=fw0, fcwt1=fw1)[0]

  def tileize(arr):
    p = jnp.pad(arr, ((0, 0), (0, _LPAD - _L)))
    return p.T.reshape(_LPAD, 32, _DPT).transpose(1, 0, 2).reshape(-1)

  ids_f = tileize(doc_word_ids.astype(jnp.int32))
  w_f = tileize(doc_weights)
  out_f = _pool(z.reshape(-1), ids_f, w_f, jnp.repeat(fcb, 16))
  return out_f.reshape(_NC, _ND).T
